# Initial kernel scaffold; baseline (speedup 1.0000x reference)
import jax
import jax.numpy as jnp
from jax.experimental import pallas as pl


def kernel(x, table):
    raise NotImplementedError("write your pallas kernel here")


# speedup vs baseline: 18.5285x; 18.5285x over previous
import jax
import jax.numpy as jnp
from jax import lax
from jax.experimental import pallas as pl
from jax.experimental.pallas import tpu as pltpu
from jax.experimental.pallas import tpu_sc as plsc

_SEMEME = 100000
_MAX_NORM = 5.0
_DIM = 32
_K = 20
_NC, _NS = 2, 16
_NW = _NC * _NS
_G = 1024 * 50
_CM = 128
_NCH = _G // _CM
_CPW = 13

_SCALE_BLK = 1024
_ROWS_PAD = 98 * _SCALE_BLK


def _scale_body(t_ref, o_ref):
    t = t_ref[...]
    ss = jnp.sum(t * t, axis=1, keepdims=True)
    norm = jnp.sqrt(ss)
    scale = jnp.minimum(1.0, _MAX_NORM / jnp.maximum(norm, 1e-7))
    o_ref[...] = t * scale


def _scale_table(table):
    return pl.pallas_call(
        _scale_body,
        grid=(_ROWS_PAD // _SCALE_BLK,),
        in_specs=[pl.BlockSpec((_SCALE_BLK, _DIM), lambda i: (i, 0))],
        out_specs=pl.BlockSpec((_SCALE_BLK, _DIM), lambda i: (i, 0)),
        out_shape=jax.ShapeDtypeStruct((_ROWS_PAD, _DIM), jnp.float32),
    )(table)


def _sc_body(table_hbm, xt_hbm, out_hbm, idx_v, acc_v, recip_v, sem0, sem1):
    wid = lax.axis_index("s") * _NC + lax.axis_index("c")
    start = (wid * (_NCH - _CPW)) // (_NW - 1)

    pltpu.sync_copy(xt_hbm.at[pl.ds(start, _CPW)], idx_v)

    def fire0(c, _):
        pltpu.async_copy(table_hbm.at[idx_v.at[c, 0]], acc_v.at[c], sem0)
        return 0

    lax.fori_loop(0, _CPW, fire0, 0)

    def drain0(c, _):
        pltpu.make_async_copy(
            table_hbm.at[idx_v.at[0, 0]], acc_v.at[0], sem0).wait()
        return 0

    lax.fori_loop(0, _CPW, drain0, 0)

    def fire_add(i, _):
        c = i // (_K - 1)
        j = i % (_K - 1) + 1
        pltpu.async_copy(
            table_hbm.at[idx_v.at[c, j]], acc_v.at[c], sem1, add=True)
        return 0

    lax.fori_loop(0, _CPW * (_K - 1), fire_add, 0)

    def cnt_body(i, _):
        c = i // (_CM // 16)
        mb = (i % (_CM // 16)) * 16

        def inner(j, s):
            v = idx_v[c, j, pl.ds(mb, 16)]
            return s + jnp.where(v < _SEMEME, 1.0, 0.0)

        s = lax.fori_loop(0, _K, inner, jnp.zeros((16,), jnp.float32))
        recip_v[c, pl.ds(mb, 16)] = 1.0 / jnp.maximum(s, 1.0)
        return 0

    lax.fori_loop(0, _CPW * (_CM // 16), cnt_body, 0)

    def drain_add(i, _):
        pltpu.make_async_copy(
            table_hbm.at[idx_v.at[0, 0]], acc_v.at[0], sem1).wait()
        return 0

    lax.fori_loop(0, _CPW * (_K - 1), drain_add, 0)

    def mul_body(i, _):
        c = i // (_CM // 16)
        mb = (i % (_CM // 16)) * 16
        r = recip_v[c, pl.ds(mb, 16)]
        for l in range(16):
            s = r[l]
            g = mb + l
            acc_v[c, g, pl.ds(0, 16)] = acc_v[c, g, pl.ds(0, 16)] * s
            acc_v[c, g, pl.ds(16, 16)] = acc_v[c, g, pl.ds(16, 16)] * s
        return 0

    lax.fori_loop(0, _CPW * (_CM // 16), mul_body, 0)

    pltpu.sync_copy(acc_v, out_hbm.at[pl.ds(start, _CPW)])


@jax.jit
def _sc_embed(scaled_table, xt):
    mesh = plsc.VectorSubcoreMesh(
        core_axis_name="c", subcore_axis_name="s",
        num_cores=_NC, num_subcores=_NS,
    )
    return pl.kernel(
        _sc_body,
        out_type=jax.ShapeDtypeStruct((_NCH, _CM, _DIM), jnp.float32),
        mesh=mesh,
        compiler_params=pltpu.CompilerParams(use_tc_tiling_on_sc=False),
        scratch_types=[
            pltpu.VMEM((_CPW, _K, _CM), jnp.int32),
            pltpu.VMEM((_CPW, _CM, _DIM), jnp.float32),
            pltpu.VMEM((_CPW, _CM), jnp.float32),
            pltpu.SemaphoreType.DMA,
            pltpu.SemaphoreType.DMA,
        ],
    )(scaled_table, xt)


def kernel(x, table):
    scaled = _scale_table(table)
    xt = x.reshape(_NCH, _CM, _K).transpose(0, 2, 1)
    out = _sc_embed(scaled, xt)
    return out.reshape(1024, 50, _DIM)


# speedup vs baseline: 20.4493x; 1.1037x over previous
import jax
import jax.numpy as jnp
from jax import lax
from jax.experimental import pallas as pl
from jax.experimental.pallas import tpu as pltpu
from jax.experimental.pallas import tpu_sc as plsc

_SEMEME = 100000
_MAX_NORM = 5.0
_DIM = 32
_K = 20
_NC, _NS = 2, 16
_NW = _NC * _NS
_G = 1024 * 50
_CM = 128
_NCH = _G // _CM
_CPW = 13

_SCALE_BLK = 8192
_SCALE_GRID = 13


def _scale_body(t_ref, o_ref):
    t = t_ref[...]
    ss = jnp.sum(t * t, axis=1, keepdims=True)
    norm = jnp.sqrt(ss)
    scale = jnp.minimum(1.0, _MAX_NORM / jnp.maximum(norm, 1e-7))
    o_ref[...] = t * scale


def _scale_table(table):
    return pl.pallas_call(
        _scale_body,
        grid=(_SCALE_GRID,),
        in_specs=[pl.BlockSpec((_SCALE_BLK, _DIM), lambda i: (i, 0))],
        out_specs=pl.BlockSpec((_SCALE_BLK, _DIM), lambda i: (i, 0)),
        out_shape=jax.ShapeDtypeStruct((_SCALE_GRID * _SCALE_BLK, _DIM),
                                       jnp.float32),
    )(table)


def _sc_body(table_hbm, x_hbm, out_hbm, x_v, idx_v, acc_v, recip_v, gsem):
    wid = lax.axis_index("s") * _NC + lax.axis_index("c")
    start = (wid * (_NCH - _CPW)) // (_NW - 1)

    pltpu.sync_copy(x_hbm.at[pl.ds(start * (_CM * _K // 128), _CPW * _CM * _K // 128)], x_v)

    stride = lax.iota(jnp.int32, 16) * _K
    zeros16 = jnp.zeros((16,), jnp.float32)

    def chunk_body(c, _):
        def mb_body(mb, _):
            base = c * (_CM * _K) + mb * (16 * _K)

            def j_body(j, s):
                off = stride + (base + j)
                v = plsc.load_gather(x_v, [off >> 7, off & 127])
                idx_v[c, j, pl.ds(mb * 16, 16)] = v
                return s + jnp.where(v < _SEMEME, 1.0, 0.0)

            s = lax.fori_loop(0, _K, j_body, zeros16, unroll=4)
            recip_v[c, pl.ds(mb * 16, 16)] = 1.0 / jnp.maximum(s, 1.0)
            return 0

        lax.fori_loop(0, _CM // 16, mb_body, 0)

        def z_body(i, _):
            acc_v[c, i, pl.ds(0, 16)] = zeros16
            acc_v[c, i, pl.ds(16, 16)] = zeros16
            return 0

        lax.fori_loop(0, _CM, z_body, 0, unroll=8)

        def fire(j, _):
            pltpu.async_copy(
                table_hbm.at[idx_v.at[c, j]], acc_v.at[c], gsem, add=True)
            return 0

        lax.fori_loop(0, _K, fire, 0)
        return 0

    lax.fori_loop(0, _CPW, chunk_body, 0)

    def drain(i, _):
        pltpu.make_async_copy(
            table_hbm.at[idx_v.at[0, 0]], acc_v.at[0], gsem).wait()
        return 0

    lax.fori_loop(0, _CPW * _K, drain, 0)

    def mul_body(i, _):
        c = i // (_CM // 16)
        mb = (i % (_CM // 16)) * 16
        r = recip_v[c, pl.ds(mb, 16)]
        for l in range(16):
            s = r[l]
            g = mb + l
            acc_v[c, g, pl.ds(0, 16)] = acc_v[c, g, pl.ds(0, 16)] * s
            acc_v[c, g, pl.ds(16, 16)] = acc_v[c, g, pl.ds(16, 16)] * s
        return 0

    lax.fori_loop(0, _CPW * (_CM // 16), mul_body, 0)

    pltpu.sync_copy(acc_v, out_hbm.at[pl.ds(start, _CPW)])


@jax.jit
def _sc_embed(scaled_table, x_flat):
    mesh = plsc.VectorSubcoreMesh(
        core_axis_name="c", subcore_axis_name="s",
        num_cores=_NC, num_subcores=_NS,
    )
    return pl.kernel(
        _sc_body,
        out_type=jax.ShapeDtypeStruct((_NCH, _CM, _DIM), jnp.float32),
        mesh=mesh,
        compiler_params=pltpu.CompilerParams(use_tc_tiling_on_sc=False,
                                             needs_layout_passes=False),
        scratch_types=[
            pltpu.VMEM((_CPW * _CM * _K // 128, 128), jnp.int32),
            pltpu.VMEM((_CPW, _K, _CM), jnp.int32),
            pltpu.VMEM((_CPW, _CM, _DIM), jnp.float32),
            pltpu.VMEM((_CPW, _CM), jnp.float32),
            pltpu.SemaphoreType.DMA,
        ],
    )(scaled_table, x_flat)


def kernel(x, table):
    scaled = _scale_table(table)
    out = _sc_embed(scaled, x.reshape(-1, 128))
    return out.reshape(1024, 50, _DIM)


# speedup vs baseline: 24.4830x; 1.1973x over previous
import jax
import jax.numpy as jnp
from jax import lax
from jax.experimental import pallas as pl
from jax.experimental.pallas import tpu as pltpu
from jax.experimental.pallas import tpu_sc as plsc

_SEMEME = 100000
_MAX_NORM = 5.0
_DIM = 32
_K = 20
_NC, _NS = 2, 16
_NW = _NC * _NS
_G = 1024 * 50
_CM = 128
_NCH = _G // _CM
_CPW = 13

_SCALE_BLK = 2048
_SCALE_GRID = 16
_Q = 32768
_LAST_BLK = 100001 // _SCALE_BLK


def _scale_body(t0_ref, t1_ref, t2_ref, t3_ref, o_ref):
    t = jnp.concatenate(
        [t0_ref[...], t1_ref[...], t2_ref[...], t3_ref[...]], axis=1)
    bi = lax.broadcasted_iota(jnp.int32, (4 * _DIM, 4 * _DIM), 0) // _DIM
    bj = lax.broadcasted_iota(jnp.int32, (4 * _DIM, 4 * _DIM), 1) // _DIM
    bd = jnp.where(bi == bj, 1.0, 0.0).astype(jnp.float32)
    ss = jnp.dot(t * t, bd, preferred_element_type=jnp.float32)
    norm = jnp.sqrt(ss)
    scale = jnp.minimum(1.0, _MAX_NORM / jnp.maximum(norm, 1e-7))
    o_ref[...] = t * scale


def _quarter_spec(k):
    def index_map(i):
        return (jnp.minimum(i + k * _SCALE_GRID, _LAST_BLK), 0)
    return pl.BlockSpec((_SCALE_BLK, _DIM), index_map)


def _scale_table(table):
    return pl.pallas_call(
        _scale_body,
        grid=(_SCALE_GRID,),
        in_specs=[_quarter_spec(k) for k in range(4)],
        out_specs=pl.BlockSpec((_SCALE_BLK, 4 * _DIM), lambda i: (i, 0)),
        out_shape=jax.ShapeDtypeStruct((_Q, 4 * _DIM), jnp.float32),
    )(table, table, table, table)


def _sc_body(table_hbm, x_hbm, out_hbm, x_v, idx_v, acc_v, recip_v, gsem):
    wid = lax.axis_index("s") * _NC + lax.axis_index("c")
    start = (wid * (_NCH - _CPW)) // (_NW - 1)

    pltpu.sync_copy(x_hbm.at[pl.ds(start * (_CM * _K // 128), _CPW * _CM * _K // 128)], x_v)

    stride = lax.iota(jnp.int32, 16) * _K
    zeros16 = jnp.zeros((16,), jnp.float32)

    def chunk_body(c, _):
        def mb_body(mb, _):
            base = c * (_CM * _K) + mb * (16 * _K)

            def j_body(j, s):
                off = stride + (base + j)
                v = plsc.load_gather(x_v, [off >> 7, off & 127])
                idx_v[c, j, pl.ds(mb * 16, 16)] = ((v & (_Q - 1)) << 2) | (v >> 15)
                return s + jnp.where(v < _SEMEME, 1.0, 0.0)

            s = lax.fori_loop(0, _K, j_body, zeros16, unroll=4)
            recip_v[c, pl.ds(mb * 16, 16)] = 1.0 / jnp.maximum(s, 1.0)
            return 0

        lax.fori_loop(0, _CM // 16, mb_body, 0)

        def z_body(i, _):
            acc_v[c, i, pl.ds(0, 16)] = zeros16
            acc_v[c, i, pl.ds(16, 16)] = zeros16
            return 0

        lax.fori_loop(0, _CM, z_body, 0, unroll=8)

        def fire(j, _):
            pltpu.async_copy(
                table_hbm.at[idx_v.at[c, j]], acc_v.at[c], gsem, add=True)
            return 0

        lax.fori_loop(0, _K, fire, 0)
        return 0

    lax.fori_loop(0, _CPW, chunk_body, 0)

    def drain(i, _):
        pltpu.make_async_copy(
            table_hbm.at[idx_v.at[0, 0]], acc_v.at[0], gsem).wait()
        return 0

    lax.fori_loop(0, _CPW * _K, drain, 0)

    def mul_body(i, _):
        c = i // (_CM // 16)
        mb = (i % (_CM // 16)) * 16
        r = recip_v[c, pl.ds(mb, 16)]
        for l in range(16):
            s = r[l]
            g = mb + l
            acc_v[c, g, pl.ds(0, 16)] = acc_v[c, g, pl.ds(0, 16)] * s
            acc_v[c, g, pl.ds(16, 16)] = acc_v[c, g, pl.ds(16, 16)] * s
        return 0

    lax.fori_loop(0, _CPW * (_CM // 16), mul_body, 0)

    pltpu.sync_copy(acc_v, out_hbm.at[pl.ds(start, _CPW)])


@jax.jit
def _sc_embed(scaled_table, x_flat):
    mesh = plsc.VectorSubcoreMesh(
        core_axis_name="c", subcore_axis_name="s",
        num_cores=_NC, num_subcores=_NS,
    )
    return pl.kernel(
        _sc_body,
        out_type=jax.ShapeDtypeStruct((_NCH, _CM, _DIM), jnp.float32),
        mesh=mesh,
        compiler_params=pltpu.CompilerParams(use_tc_tiling_on_sc=False,
                                             needs_layout_passes=False),
        scratch_types=[
            pltpu.VMEM((_CPW * _CM * _K // 128, 128), jnp.int32),
            pltpu.VMEM((_CPW, _K, _CM), jnp.int32),
            pltpu.VMEM((_CPW, _CM, _DIM), jnp.float32),
            pltpu.VMEM((_CPW, _CM), jnp.float32),
            pltpu.SemaphoreType.DMA,
        ],
    )(scaled_table, x_flat)


def kernel(x, table):
    scaled = _scale_table(table).reshape(-1, _DIM)
    out = _sc_embed(scaled, x.reshape(-1, 128))
    return out.reshape(1024, 50, _DIM)


# speedup vs baseline: 27.5590x; 1.1256x over previous
import jax
import jax.numpy as jnp
from jax import lax
from jax.experimental import pallas as pl
from jax.experimental.pallas import tpu as pltpu
from jax.experimental.pallas import tpu_sc as plsc

_SEMEME = 100000
_MAX_NORM = 5.0
_DIM = 32
_K = 20
_NC, _NS = 2, 16
_NW = _NC * _NS
_G = 1024 * 50
_CM = 128
_NCH = _G // _CM
_CPW = 13

_SCALE_BLK = 2048
_SCALE_GRID = 16
_Q = 32768
_LAST_BLK = 100001 // _SCALE_BLK


def _scale_body(t0_ref, t1_ref, t2_ref, t3_ref, o_ref):
    t = jnp.concatenate(
        [t0_ref[...], t1_ref[...], t2_ref[...], t3_ref[...]], axis=1)
    bi = lax.broadcasted_iota(jnp.int32, (4 * _DIM, 4 * _DIM), 0) // _DIM
    bj = lax.broadcasted_iota(jnp.int32, (4 * _DIM, 4 * _DIM), 1) // _DIM
    bd = jnp.where(bi == bj, 1.0, 0.0).astype(jnp.float32)
    ss = jnp.dot(t * t, bd, preferred_element_type=jnp.float32)
    norm = jnp.sqrt(ss)
    scale = jnp.minimum(1.0, _MAX_NORM / jnp.maximum(norm, 1e-7))
    o_ref[...] = t * scale


def _quarter_spec(k):
    def index_map(i):
        return (jnp.minimum(i + k * _SCALE_GRID, _LAST_BLK), 0)
    return pl.BlockSpec((_SCALE_BLK, _DIM), index_map)


def _scale_table(table):
    return pl.pallas_call(
        _scale_body,
        grid=(_SCALE_GRID,),
        in_specs=[_quarter_spec(k) for k in range(4)],
        out_specs=pl.BlockSpec((_SCALE_BLK, 4 * _DIM), lambda i: (i, 0)),
        out_shape=jax.ShapeDtypeStruct((_Q, 4 * _DIM), jnp.float32),
    )(table, table, table, table)


def _sc_body(table_hbm, x_hbm, out_hbm, x_v, idx_v, acc_v, recip_v, slab_v,
             gsem, osem):
    wid = lax.axis_index("s") * _NC + lax.axis_index("c")
    b0 = wid * 32

    pltpu.sync_copy(x_hbm.at[pl.ds(wid * 250, 250)], x_v)

    stride = lax.iota(jnp.int32, 16) * _K
    zeros16 = jnp.zeros((16,), jnp.float32)

    def chunk_body(c, _):
        def mb_body(mb, _):
            base = c * (_CM * _K) + mb * (16 * _K)

            def j_body(j, s):
                off = stride + (base + j)
                v = plsc.load_gather(x_v, [off >> 7, off & 127])
                idx_v[c, j, pl.ds(mb * 16, 16)] = (
                    ((v & (_Q - 1)) << 2) | ((v >> 15) & 3))
                return s + jnp.where(v < _SEMEME, 1.0, 0.0)

            s = lax.fori_loop(0, _K, j_body, zeros16, unroll=4)
            recip_v[c, pl.ds(mb * 16, 16)] = 1.0 / jnp.maximum(s, 1.0)
            return 0

        lax.fori_loop(0, _CM // 16, mb_body, 0)

        def z_body(i, _):
            acc_v[c, i, pl.ds(0, 16)] = zeros16
            acc_v[c, i, pl.ds(16, 16)] = zeros16
            return 0

        lax.fori_loop(0, _CM, z_body, 0, unroll=8)

        def fire(j, _):
            pltpu.async_copy(
                table_hbm.at[idx_v.at[c, j]], acc_v.at[c], gsem, add=True)
            return 0

        lax.fori_loop(0, _K, fire, 0)
        return 0

    lax.fori_loop(0, _CPW, chunk_body, 0)

    def drain(i, _):
        pltpu.make_async_copy(
            table_hbm.at[idx_v.at[0, 0]], acc_v.at[0], gsem).wait()
        return 0

    lax.fori_loop(0, _CPW * _K, drain, 0)

    iota16 = lax.iota(jnp.int32, 16)

    def s_body(s, _):
        sp = s & 1

        @pl.when(s >= 2)
        def _wait_slab():
            pltpu.make_async_copy(
                slab_v.at[0], out_hbm.at[0, :, pl.ds(0, 32)], osem).wait()

        q0 = iota16 * 50 + s
        r0 = plsc.load_gather(recip_v, [q0 >> 7, q0 & 127])
        q1 = q0 + 800
        r1 = plsc.load_gather(recip_v, [q1 >> 7, q1 & 127])

        def bi_body(bi, _):
            q = bi * 50 + s
            c = q >> 7
            m = q & 127
            col = jnp.broadcast_to(bi, (16,)).astype(jnp.int32)
            plsc.store_scatter(slab_v.at[sp], [iota16, col],
                               acc_v[c, m, pl.ds(0, 16)])
            plsc.store_scatter(slab_v.at[sp], [iota16 + 16, col],
                               acc_v[c, m, pl.ds(16, 16)])
            return 0

        lax.fori_loop(0, 32, bi_body, 0)

        def d_body(d, _):
            slab_v[sp, d, pl.ds(0, 16)] = slab_v[sp, d, pl.ds(0, 16)] * r0
            slab_v[sp, d, pl.ds(16, 16)] = slab_v[sp, d, pl.ds(16, 16)] * r1
            return 0

        lax.fori_loop(0, _DIM, d_body, 0, unroll=4)

        pltpu.async_copy(slab_v.at[sp], out_hbm.at[s, :, pl.ds(b0, 32)], osem)
        return 0

    lax.fori_loop(0, 50, s_body, 0)

    pltpu.make_async_copy(
        slab_v.at[0], out_hbm.at[0, :, pl.ds(0, 32)], osem).wait()
    pltpu.make_async_copy(
        slab_v.at[0], out_hbm.at[0, :, pl.ds(0, 32)], osem).wait()


@jax.jit
def _sc_embed(scaled_table, x_flat):
    mesh = plsc.VectorSubcoreMesh(
        core_axis_name="c", subcore_axis_name="s",
        num_cores=_NC, num_subcores=_NS,
    )
    return pl.kernel(
        _sc_body,
        out_type=jax.ShapeDtypeStruct((50, _DIM, 1024), jnp.float32),
        mesh=mesh,
        compiler_params=pltpu.CompilerParams(use_tc_tiling_on_sc=False,
                                             needs_layout_passes=False),
        scratch_types=[
            pltpu.VMEM((250, 128), jnp.int32),
            pltpu.VMEM((_CPW, _K, _CM), jnp.int32),
            pltpu.VMEM((_CPW, _CM, _DIM), jnp.float32),
            pltpu.VMEM((_CPW, _CM), jnp.float32),
            pltpu.VMEM((2, _DIM, 32), jnp.float32),
            pltpu.SemaphoreType.DMA,
            pltpu.SemaphoreType.DMA,
        ],
    )(scaled_table, x_flat)


def kernel(x, table):
    scaled = _scale_table(table).reshape(-1, _DIM)
    out = _sc_embed(scaled, x.reshape(-1, 128))
    return out.transpose(2, 0, 1)


# speedup vs baseline: 28.4610x; 1.0327x over previous
import jax
import jax.numpy as jnp
from jax import lax
from jax.experimental import pallas as pl
from jax.experimental.pallas import tpu as pltpu
from jax.experimental.pallas import tpu_sc as plsc

_SEMEME = 100000
_MAX_NORM = 5.0
_DIM = 32
_K = 20
_NC, _NS = 2, 16
_NW = _NC * _NS
_G = 1024 * 50
_CM = 128
_NCH = _G // _CM
_CPW = 13

_SCALE_BLK = 2048
_SCALE_GRID = 16
_Q = 32768
_LAST_BLK = 100001 // _SCALE_BLK


def _scale_body(t0_ref, t1_ref, t2_ref, t3_ref, o_ref):
    t = jnp.concatenate(
        [t0_ref[...], t1_ref[...], t2_ref[...], t3_ref[...]], axis=1)
    bi = lax.broadcasted_iota(jnp.int32, (4 * _DIM, 4 * _DIM), 0) // _DIM
    bj = lax.broadcasted_iota(jnp.int32, (4 * _DIM, 4 * _DIM), 1) // _DIM
    bd = jnp.where(bi == bj, 1.0, 0.0).astype(jnp.float32)
    ss = jnp.dot(t * t, bd, preferred_element_type=jnp.float32)
    norm = jnp.sqrt(ss)
    scale = jnp.minimum(1.0, _MAX_NORM / jnp.maximum(norm, 1e-7))
    o_ref[...] = t * scale


def _quarter_spec(k):
    def index_map(i):
        return (jnp.minimum(i + k * _SCALE_GRID, _LAST_BLK), 0)
    return pl.BlockSpec((_SCALE_BLK, _DIM), index_map)


def _scale_table(table):
    return pl.pallas_call(
        _scale_body,
        grid=(_SCALE_GRID,),
        in_specs=[_quarter_spec(k) for k in range(4)],
        out_specs=pl.BlockSpec((_SCALE_BLK, 4 * _DIM), lambda i: (i, 0)),
        out_shape=jax.ShapeDtypeStruct((_Q, 4 * _DIM), jnp.float32),
    )(table, table, table, table)


def _sc_body(table_hbm, x_hbm, out_hbm, x_v, idx_v, acc_v, recip_v, slab_v,
             gsem, osem):
    wid = lax.axis_index("s") * _NC + lax.axis_index("c")
    b0 = wid * 32

    pltpu.sync_copy(x_hbm.at[pl.ds(wid * 250, 250)], x_v)

    stride = lax.iota(jnp.int32, 16) * (50 * _K)
    zeros16 = jnp.zeros((16,), jnp.float32)
    iota16 = lax.iota(jnp.int32, 16)

    def chunk_body(c, _):
        def mb_body(mb, _):
            s_id = (c * _CM + mb * 16) >> 5
            bi0 = (mb * 16) & 31

            def j_body(j, s):
                off = stride + (bi0 * (50 * _K) + s_id * _K + j)
                v = plsc.load_gather(x_v, [off >> 7, off & 127])
                idx_v[c, j, pl.ds(mb * 16, 16)] = (
                    ((v & (_Q - 1)) << 2) | ((v >> 15) & 3))
                return s + jnp.where(v < _SEMEME, 1.0, 0.0)

            s = lax.fori_loop(0, _K, j_body, zeros16, unroll=4)
            recip_v[c, pl.ds(mb * 16, 16)] = 1.0 / jnp.maximum(s, 1.0)
            return 0

        lax.fori_loop(0, _CM // 16, mb_body, 0)

        def z_body(i, _):
            acc_v[c, i, pl.ds(0, 16)] = zeros16
            acc_v[c, i, pl.ds(16, 16)] = zeros16
            return 0

        lax.fori_loop(0, _CM, z_body, 0, unroll=8)

        def fire(j, _):
            pltpu.async_copy(
                table_hbm.at[idx_v.at[c, j]], acc_v.at[c], gsem.at[c],
                add=True)
            return 0

        lax.fori_loop(0, _K, fire, 0)
        return 0

    lax.fori_loop(0, _CPW, chunk_body, 0)

    def out_chunk(c, _):
        def dr(i, _):
            pltpu.make_async_copy(
                table_hbm.at[idx_v.at[0, 0]], acc_v.at[0], gsem.at[c]).wait()
            return 0

        lax.fori_loop(0, _K, dr, 0)

        def s_b(si, _):
            s = c * 4 + si

            @pl.when(s < 50)
            def _emit():
                sp = s & 3

                @pl.when(s >= 4)
                def _wait_slab():
                    pltpu.make_async_copy(
                        slab_v.at[0], out_hbm.at[0, :, pl.ds(0, 32)],
                        osem).wait()

                m0 = (s & 3) * 32
                r0 = recip_v[c, pl.ds(m0, 16)]
                r1 = recip_v[c, pl.ds(m0 + 16, 16)]

                def bi_body(bi, _):
                    m = m0 + bi
                    col = jnp.broadcast_to(bi, (16,)).astype(jnp.int32)
                    plsc.store_scatter(slab_v.at[sp], [iota16, col],
                                       acc_v[c, m, pl.ds(0, 16)])
                    plsc.store_scatter(slab_v.at[sp], [iota16 + 16, col],
                                       acc_v[c, m, pl.ds(16, 16)])
                    return 0

                lax.fori_loop(0, 32, bi_body, 0)

                def d_body(d, _):
                    slab_v[sp, d, pl.ds(0, 16)] = (
                        slab_v[sp, d, pl.ds(0, 16)] * r0)
                    slab_v[sp, d, pl.ds(16, 16)] = (
                        slab_v[sp, d, pl.ds(16, 16)] * r1)
                    return 0

                lax.fori_loop(0, _DIM, d_body, 0, unroll=4)

                pltpu.async_copy(
                    slab_v.at[sp], out_hbm.at[s, :, pl.ds(b0, 32)], osem)

            return 0

        lax.fori_loop(0, 4, s_b, 0)
        return 0

    lax.fori_loop(0, _CPW, out_chunk, 0)

    def drain_slabs(i, _):
        pltpu.make_async_copy(
            slab_v.at[0], out_hbm.at[0, :, pl.ds(0, 32)], osem).wait()
        return 0

    lax.fori_loop(0, 4, drain_slabs, 0)


@jax.jit
def _sc_embed(scaled_table, x_flat):
    mesh = plsc.VectorSubcoreMesh(
        core_axis_name="c", subcore_axis_name="s",
        num_cores=_NC, num_subcores=_NS,
    )
    return pl.kernel(
        _sc_body,
        out_type=jax.ShapeDtypeStruct((50, _DIM, 1024), jnp.float32),
        mesh=mesh,
        compiler_params=pltpu.CompilerParams(use_tc_tiling_on_sc=False,
                                             needs_layout_passes=False),
        scratch_types=[
            pltpu.VMEM((250, 128), jnp.int32),
            pltpu.VMEM((_CPW, _K, _CM), jnp.int32),
            pltpu.VMEM((_CPW, _CM, _DIM), jnp.float32),
            pltpu.VMEM((_CPW, _CM), jnp.float32),
            pltpu.VMEM((4, _DIM, 32), jnp.float32),
            pltpu.SemaphoreType.DMA((_CPW,)),
            pltpu.SemaphoreType.DMA,
        ],
    )(scaled_table, x_flat)


def kernel(x, table):
    scaled = _scale_table(table).reshape(-1, _DIM)
    out = _sc_embed(scaled, x.reshape(-1, 128))
    return out.transpose(2, 0, 1)


# speedup vs baseline: 29.6713x; 1.0425x over previous
import jax
import jax.numpy as jnp
from jax import lax
from jax.experimental import pallas as pl
from jax.experimental.pallas import tpu as pltpu
from jax.experimental.pallas import tpu_sc as plsc

_SEMEME = 100000
_MAX_NORM = 5.0
_DIM = 32
_K = 20
_NC, _NS = 2, 16
_NW = _NC * _NS
_G = 1024 * 50
_CM = 128
_NCH = _G // _CM
_CPW = 13

_SCALE_BLK = 2048
_SCALE_GRID = 16
_Q = 32768
_LAST_BLK = 100001 // _SCALE_BLK


def _scale_body(t0_ref, t1_ref, t2_ref, t3_ref, o_ref):
    t = jnp.concatenate(
        [t0_ref[...], t1_ref[...], t2_ref[...], t3_ref[...]], axis=1)
    bi = lax.broadcasted_iota(jnp.int32, (4 * _DIM, 4 * _DIM), 0) // _DIM
    bj = lax.broadcasted_iota(jnp.int32, (4 * _DIM, 4 * _DIM), 1) // _DIM
    bd = jnp.where(bi == bj, 1.0, 0.0).astype(jnp.float32)
    ss = jnp.dot(t * t, bd, preferred_element_type=jnp.float32)
    norm = jnp.sqrt(ss)
    scale = jnp.minimum(1.0, _MAX_NORM / jnp.maximum(norm, 1e-7))
    o_ref[...] = t * scale


def _quarter_spec(k):
    def index_map(i):
        return (jnp.minimum(i + k * _SCALE_GRID, _LAST_BLK), 0)
    return pl.BlockSpec((_SCALE_BLK, _DIM), index_map)


def _scale_table(table):
    return pl.pallas_call(
        _scale_body,
        grid=(_SCALE_GRID,),
        in_specs=[_quarter_spec(k) for k in range(4)],
        out_specs=pl.BlockSpec((_SCALE_BLK, 4 * _DIM), lambda i: (i, 0)),
        out_shape=jax.ShapeDtypeStruct((_Q, 4 * _DIM), jnp.float32),
    )(table, table, table, table)


def _sc_body(table_hbm, x_hbm, out_hbm, x_v, idx_v, acc_v, recip_v, slab_v,
             gsem, osem):
    wid = lax.axis_index("s") * _NC + lax.axis_index("c")
    b0 = wid * 32

    pltpu.sync_copy(x_hbm.at[pl.ds(wid * 250, 250)], x_v)

    stride = lax.iota(jnp.int32, 16) * (50 * _K)
    zeros16 = jnp.zeros((16,), jnp.float32)
    iota16 = lax.iota(jnp.int32, 16)

    def chunk_body(c, _):
        def mb_body(mb, _):
            s_id = (c * _CM + mb * 16) >> 5
            bi0 = (mb * 16) & 31

            def j_body(j, s):
                off = stride + (bi0 * (50 * _K) + s_id * _K + j)
                v = plsc.load_gather(x_v, [off >> 7, off & 127])
                idx_v[c, j, pl.ds(mb * 16, 16)] = (
                    ((v & (_Q - 1)) << 2) | ((v >> 15) & 3))
                return s + jnp.where(v < _SEMEME, 1.0, 0.0)

            s = lax.fori_loop(0, _K, j_body, zeros16, unroll=4)
            recip_v[c, pl.ds(mb * 16, 16)] = 1.0 / jnp.maximum(s, 1.0)
            return 0

        lax.fori_loop(0, _CM // 16, mb_body, 0)

        def z_body(i, _):
            acc_v[c, i, pl.ds(0, 16)] = zeros16
            acc_v[c, i, pl.ds(16, 16)] = zeros16
            return 0

        lax.fori_loop(0, _CM, z_body, 0, unroll=8)

        def fire(j, _):
            pltpu.async_copy(
                table_hbm.at[idx_v.at[c, j]], acc_v.at[c], gsem.at[c],
                add=True)
            return 0

        lax.fori_loop(0, _K, fire, 0)
        return 0

    lax.fori_loop(0, _CPW, chunk_body, 0)

    def out_chunk(c, _):
        def dr(i, _):
            pltpu.make_async_copy(
                table_hbm.at[idx_v.at[0, 0]], acc_v.at[0], gsem.at[c]).wait()
            return 0

        lax.fori_loop(0, _K, dr, 0)

        def s_b(si, _):
            s = c * 4 + si

            @pl.when(s < 50)
            def _emit():
                sp = s & 3

                @pl.when(s >= 4)
                def _wait_slab():
                    pltpu.make_async_copy(
                        slab_v.at[0, :, pl.ds(0, 32)],
                        out_hbm.at[0, :, pl.ds(0, 32)], osem).wait()

                m0 = (s & 3) * 32
                r0 = recip_v[c, pl.ds(m0, 16)]
                r1 = recip_v[c, pl.ds(m0 + 16, 16)]

                def bi_body(bi, _):
                    m = m0 + bi
                    col = jnp.broadcast_to(bi, (16,)).astype(jnp.int32)
                    plsc.store_scatter(slab_v.at[sp], [iota16, col],
                                       acc_v[c, m, pl.ds(0, 16)])
                    plsc.store_scatter(slab_v.at[sp], [iota16 + 16, col],
                                       acc_v[c, m, pl.ds(16, 16)])
                    return 0

                lax.fori_loop(0, 32, bi_body, 0)

                def d_body(d, _):
                    slab_v[sp, d, pl.ds(0, 16)] = (
                        slab_v[sp, d, pl.ds(0, 16)] * r0)
                    slab_v[sp, d, pl.ds(16, 16)] = (
                        slab_v[sp, d, pl.ds(16, 16)] * r1)
                    return 0

                lax.fori_loop(0, _DIM, d_body, 0, unroll=4)

                pltpu.async_copy(
                    slab_v.at[sp, :, pl.ds(0, 32)],
                    out_hbm.at[s, :, pl.ds(b0, 32)], osem)

            return 0

        lax.fori_loop(0, 4, s_b, 0)
        return 0

    lax.fori_loop(0, _CPW, out_chunk, 0)

    def drain_slabs(i, _):
        pltpu.make_async_copy(
            slab_v.at[0, :, pl.ds(0, 32)],
            out_hbm.at[0, :, pl.ds(0, 32)], osem).wait()
        return 0

    lax.fori_loop(0, 4, drain_slabs, 0)


@jax.jit
def _sc_embed(scaled_table, x_flat):
    mesh = plsc.VectorSubcoreMesh(
        core_axis_name="c", subcore_axis_name="s",
        num_cores=_NC, num_subcores=_NS,
    )
    return pl.kernel(
        _sc_body,
        out_type=jax.ShapeDtypeStruct((50, _DIM, 1024), jnp.float32),
        mesh=mesh,
        compiler_params=pltpu.CompilerParams(use_tc_tiling_on_sc=False,
                                             needs_layout_passes=False),
        scratch_types=[
            pltpu.VMEM((250, 128), jnp.int32),
            pltpu.VMEM((_CPW, _K, _CM), jnp.int32),
            pltpu.VMEM((_CPW, _CM, _DIM), jnp.float32),
            pltpu.VMEM((_CPW, _CM), jnp.float32),
            pltpu.VMEM((4, _DIM, 33), jnp.float32),
            pltpu.SemaphoreType.DMA((_CPW,)),
            pltpu.SemaphoreType.DMA,
        ],
    )(scaled_table, x_flat)


def kernel(x, table):
    scaled = _scale_table(table).reshape(-1, _DIM)
    out = _sc_embed(scaled, x.reshape(-1, 128))
    return out.transpose(2, 0, 1)


# speedup vs baseline: 35.1050x; 1.1831x over previous
import jax
import jax.numpy as jnp
from jax import lax
from jax.experimental import pallas as pl
from jax.experimental.pallas import tpu as pltpu
from jax.experimental.pallas import tpu_sc as plsc

_SEMEME = 100000
_MAX_NORM = 5.0
_DIM = 32
_K = 20
_NC, _NS = 2, 16
_NW = _NC * _NS
_G = 1024 * 50
_CM = 128
_NCH = _G // _CM
_CPW = 13

_SCALE_BLK = 2048
_SCALE_GRID = 16
_Q = 32768
_LAST_BLK = 100001 // _SCALE_BLK


def _scale_body(t0_ref, t1_ref, t2_ref, t3_ref, o_ref):
    t = jnp.concatenate(
        [t0_ref[...], t1_ref[...], t2_ref[...], t3_ref[...]], axis=1)
    bi = lax.broadcasted_iota(jnp.int32, (4 * _DIM, 4 * _DIM), 0) // _DIM
    bj = lax.broadcasted_iota(jnp.int32, (4 * _DIM, 4 * _DIM), 1) // _DIM
    bd = jnp.where(bi == bj, 1.0, 0.0).astype(jnp.float32)
    ss = jnp.dot(t * t, bd, preferred_element_type=jnp.float32)
    norm = jnp.sqrt(ss)
    scale = jnp.minimum(1.0, _MAX_NORM / jnp.maximum(norm, 1e-7))
    o_ref[...] = t * scale


def _quarter_spec(k):
    def index_map(i):
        return (jnp.minimum(i + k * _SCALE_GRID, _LAST_BLK), 0)
    return pl.BlockSpec((_SCALE_BLK, _DIM), index_map)


def _scale_table(table):
    return pl.pallas_call(
        _scale_body,
        grid=(_SCALE_GRID,),
        in_specs=[_quarter_spec(k) for k in range(4)],
        out_specs=pl.BlockSpec((_SCALE_BLK, 4 * _DIM), lambda i: (i, 0)),
        out_shape=jax.ShapeDtypeStruct((_Q, 4 * _DIM), jnp.float32),
    )(table, table, table, table)


def _sc_body(table_hbm, x_hbm, out_hbm, xs_v, idx_v, acc_v, recip_v, slab_v,
             ssem, gsem, osem):
    wid = lax.axis_index("s") * _NC + lax.axis_index("c")
    b0 = wid * 32

    zeros16 = jnp.zeros((16,), jnp.float32)
    iota16 = lax.iota(jnp.int32, 16)

    def stage(c, buf):
        pltpu.async_copy(
            x_hbm.at[pl.ds(jnp.minimum(4 * c, 46), 4), :, pl.ds(b0, 32)],
            xs_v.at[buf], ssem)

    stage(0, 0)

    def chunk_body(c, _):
        par = c & 1
        pltpu.make_async_copy(
            x_hbm.at[pl.ds(0, 4), :, pl.ds(0, 32)], xs_v.at[0], ssem).wait()

        @pl.when(c + 1 < _CPW)
        def _next_stage():
            stage(c + 1, (c + 1) & 1)

        def mb_body(mb, _):
            sl = mb >> 1
            bi0 = (mb & 1) * 16

            def j_body(j, s):
                v = xs_v[par, sl, j, pl.ds(bi0, 16)]
                idx_v[c, j, pl.ds(mb * 16, 16)] = (
                    ((v & (_Q - 1)) << 2) | ((v >> 15) & 3))
                return s + jnp.where(v < _SEMEME, 1.0, 0.0)

            s = lax.fori_loop(0, _K, j_body, zeros16, unroll=4)
            recip_v[c, pl.ds(mb * 16, 16)] = 1.0 / jnp.maximum(s, 1.0)
            return 0

        lax.fori_loop(0, _CM // 16, mb_body, 0)

        def z_body(i, _):
            acc_v[c, i, pl.ds(0, 16)] = zeros16
            acc_v[c, i, pl.ds(16, 16)] = zeros16
            return 0

        lax.fori_loop(0, _CM, z_body, 0, unroll=8)

        def fire(j, _):
            pltpu.async_copy(
                table_hbm.at[idx_v.at[c, j]], acc_v.at[c], gsem.at[c],
                add=True)
            return 0

        lax.fori_loop(0, _K, fire, 0)
        return 0

    lax.fori_loop(0, _CPW, chunk_body, 0)

    def out_chunk(c, _):
        def dr(i, _):
            pltpu.make_async_copy(
                table_hbm.at[idx_v.at[0, 0]], acc_v.at[0], gsem.at[c]).wait()
            return 0

        lax.fori_loop(0, _K, dr, 0)

        def s_b(si, _):
            s = c * 4 + si

            @pl.when(s < 50)
            def _emit():
                sp = s & 3

                @pl.when(s >= 4)
                def _wait_slab():
                    pltpu.make_async_copy(
                        slab_v.at[0, :, pl.ds(0, 32)],
                        out_hbm.at[0, :, pl.ds(0, 32)], osem).wait()

                m0 = (s & 3) * 32 + jnp.where(c == _CPW - 1, 64, 0)
                r0 = recip_v[c, pl.ds(m0, 16)]
                r1 = recip_v[c, pl.ds(m0 + 16, 16)]

                def bi_body(bi, _):
                    m = m0 + bi
                    col = jnp.broadcast_to(bi, (16,)).astype(jnp.int32)
                    plsc.store_scatter(slab_v.at[sp], [iota16, col],
                                       acc_v[c, m, pl.ds(0, 16)])
                    plsc.store_scatter(slab_v.at[sp], [iota16 + 16, col],
                                       acc_v[c, m, pl.ds(16, 16)])
                    return 0

                lax.fori_loop(0, 32, bi_body, 0)

                def d_body(d, _):
                    slab_v[sp, d, pl.ds(0, 16)] = (
                        slab_v[sp, d, pl.ds(0, 16)] * r0)
                    slab_v[sp, d, pl.ds(16, 16)] = (
                        slab_v[sp, d, pl.ds(16, 16)] * r1)
                    return 0

                lax.fori_loop(0, _DIM, d_body, 0, unroll=4)

                pltpu.async_copy(
                    slab_v.at[sp, :, pl.ds(0, 32)],
                    out_hbm.at[s, :, pl.ds(b0, 32)], osem)

            return 0

        lax.fori_loop(0, 4, s_b, 0)
        return 0

    lax.fori_loop(0, _CPW, out_chunk, 0)

    def drain_slabs(i, _):
        pltpu.make_async_copy(
            slab_v.at[0, :, pl.ds(0, 32)],
            out_hbm.at[0, :, pl.ds(0, 32)], osem).wait()
        return 0

    lax.fori_loop(0, 4, drain_slabs, 0)


@jax.jit
def _sc_embed(scaled_table, x_flat):
    mesh = plsc.VectorSubcoreMesh(
        core_axis_name="c", subcore_axis_name="s",
        num_cores=_NC, num_subcores=_NS,
    )
    return pl.kernel(
        _sc_body,
        out_type=jax.ShapeDtypeStruct((50, _DIM, 1024), jnp.float32),
        mesh=mesh,
        compiler_params=pltpu.CompilerParams(use_tc_tiling_on_sc=False,
                                             needs_layout_passes=False),
        scratch_types=[
            pltpu.VMEM((2, 4, _K, 32), jnp.int32),
            pltpu.VMEM((_CPW, _K, _CM), jnp.int32),
            pltpu.VMEM((_CPW, _CM, _DIM), jnp.float32),
            pltpu.VMEM((_CPW, _CM), jnp.float32),
            pltpu.VMEM((4, _DIM, 33), jnp.float32),
            pltpu.SemaphoreType.DMA,
            pltpu.SemaphoreType.DMA((_CPW,)),
            pltpu.SemaphoreType.DMA,
        ],
    )(scaled_table, x_flat)


def kernel(x, table):
    scaled = _scale_table(table).reshape(-1, _DIM)
    out = _sc_embed(scaled, x.transpose(1, 2, 0))
    return out.transpose(2, 0, 1)


# speedup vs baseline: 42.3455x; 1.2063x over previous
import jax
import jax.numpy as jnp
from jax import lax
from jax.experimental import pallas as pl
from jax.experimental.pallas import tpu as pltpu
from jax.experimental.pallas import tpu_sc as plsc

_SEMEME = 100000
_MAX_NORM = 5.0
_DIM = 32
_K = 20
_NC, _NS = 2, 16
_NW = _NC * _NS
_G = 1024 * 50
_CM = 128
_NCH = _G // _CM
_CPW = 13

_SCALE_BLK = 8192
_Q = 32768
_LAST_BLK = 100001 // _SCALE_BLK


def _scale_body(t0_ref, t1_ref, t2_ref, t3_ref, o_ref):
    cols = []
    for t_ref in (t0_ref, t1_ref, t2_ref, t3_ref):
        t = t_ref[...]
        ss = jnp.sum(t * t, axis=0, keepdims=True)
        scale = jnp.minimum(1.0,
                            _MAX_NORM / jnp.maximum(jnp.sqrt(ss), 1e-7))
        cols.append((t * scale).T)
    o_ref[...] = jnp.concatenate(cols, axis=1)


def _quarter_spec(k):
    def index_map(i):
        return (0, jnp.minimum(i + k * 4, _LAST_BLK))
    return pl.BlockSpec((_DIM, _SCALE_BLK), index_map)


def _scale_table(table):
    tt = table.transpose(1, 0)
    return pl.pallas_call(
        _scale_body,
        grid=(4,),
        in_specs=[_quarter_spec(k) for k in range(4)],
        out_specs=pl.BlockSpec((_SCALE_BLK, 4 * _DIM), lambda i: (i, 0)),
        out_shape=jax.ShapeDtypeStruct((_Q, 4 * _DIM), jnp.float32),
    )(tt, tt, tt, tt)


def _sc_body(table_hbm, x_hbm, out_hbm, xs_v, idx_v, acc_v, recip_v, slab_v,
             ssem, gsem, osem):
    wid = lax.axis_index("s") * _NC + lax.axis_index("c")
    b0 = wid * 32

    zeros16 = jnp.zeros((16,), jnp.float32)
    iota16 = lax.iota(jnp.int32, 16)

    def stage(c, buf):
        pltpu.async_copy(
            x_hbm.at[pl.ds(jnp.minimum(4 * c, 46), 4), :, pl.ds(b0, 32)],
            xs_v.at[buf], ssem)

    stage(0, 0)

    def chunk_body(c, _):
        par = c & 1
        pltpu.make_async_copy(
            x_hbm.at[pl.ds(0, 4), :, pl.ds(0, 32)], xs_v.at[0], ssem).wait()

        @pl.when(c + 1 < _CPW)
        def _next_stage():
            stage(c + 1, (c + 1) & 1)

        def mb_body(mb, _):
            sl = mb >> 1
            bi0 = (mb & 1) * 16

            def j_body(j, s):
                v = xs_v[par, sl, j, pl.ds(bi0, 16)]
                idx_v[c, j, pl.ds(mb * 16, 16)] = (
                    ((v & (_Q - 1)) << 2) | ((v >> 15) & 3))
                return s + jnp.where(v < _SEMEME, 1.0, 0.0)

            s = lax.fori_loop(0, _K, j_body, zeros16, unroll=4)
            recip_v[c, pl.ds(mb * 16, 16)] = 1.0 / jnp.maximum(s, 1.0)
            return 0

        lax.fori_loop(0, _CM // 16, mb_body, 0)

        def z_body(i, _):
            acc_v[c, i, pl.ds(0, 16)] = zeros16
            acc_v[c, i, pl.ds(16, 16)] = zeros16
            return 0

        lax.fori_loop(0, _CM, z_body, 0, unroll=8)

        def fire(j, _):
            pltpu.async_copy(
                table_hbm.at[idx_v.at[c, j]], acc_v.at[c], gsem.at[c],
                add=True)
            return 0

        lax.fori_loop(0, _K, fire, 0)
        return 0

    lax.fori_loop(0, _CPW, chunk_body, 0)

    def out_chunk(c, _):
        def dr(i, _):
            pltpu.make_async_copy(
                table_hbm.at[idx_v.at[0, 0]], acc_v.at[0], gsem.at[c]).wait()
            return 0

        lax.fori_loop(0, _K, dr, 0)

        def s_b(si, _):
            s = c * 4 + si

            @pl.when(s < 50)
            def _emit():
                sp = s & 3

                @pl.when(s >= 4)
                def _wait_slab():
                    pltpu.make_async_copy(
                        slab_v.at[0, :, pl.ds(0, 32)],
                        out_hbm.at[0, :, pl.ds(0, 32)], osem).wait()

                m0 = (s & 3) * 32 + jnp.where(c == _CPW - 1, 64, 0)
                r0 = recip_v[c, pl.ds(m0, 16)]
                r1 = recip_v[c, pl.ds(m0 + 16, 16)]

                def bi_body(bi, _):
                    m = m0 + bi
                    col = jnp.broadcast_to(bi, (16,)).astype(jnp.int32)
                    plsc.store_scatter(slab_v.at[sp], [iota16, col],
                                       acc_v[c, m, pl.ds(0, 16)])
                    plsc.store_scatter(slab_v.at[sp], [iota16 + 16, col],
                                       acc_v[c, m, pl.ds(16, 16)])
                    return 0

                lax.fori_loop(0, 32, bi_body, 0)

                def d_body(d, _):
                    slab_v[sp, d, pl.ds(0, 16)] = (
                        slab_v[sp, d, pl.ds(0, 16)] * r0)
                    slab_v[sp, d, pl.ds(16, 16)] = (
                        slab_v[sp, d, pl.ds(16, 16)] * r1)
                    return 0

                lax.fori_loop(0, _DIM, d_body, 0, unroll=4)

                pltpu.async_copy(
                    slab_v.at[sp, :, pl.ds(0, 32)],
                    out_hbm.at[s, :, pl.ds(b0, 32)], osem)

            return 0

        lax.fori_loop(0, 4, s_b, 0)
        return 0

    lax.fori_loop(0, _CPW, out_chunk, 0)

    def drain_slabs(i, _):
        pltpu.make_async_copy(
            slab_v.at[0, :, pl.ds(0, 32)],
            out_hbm.at[0, :, pl.ds(0, 32)], osem).wait()
        return 0

    lax.fori_loop(0, 4, drain_slabs, 0)


@jax.jit
def _sc_embed(scaled_table, x_flat):
    mesh = plsc.VectorSubcoreMesh(
        core_axis_name="c", subcore_axis_name="s",
        num_cores=_NC, num_subcores=_NS,
    )
    return pl.kernel(
        _sc_body,
        out_type=jax.ShapeDtypeStruct((50, _DIM, 1024), jnp.float32),
        mesh=mesh,
        compiler_params=pltpu.CompilerParams(use_tc_tiling_on_sc=False,
                                             needs_layout_passes=False),
        scratch_types=[
            pltpu.VMEM((2, 4, _K, 32), jnp.int32),
            pltpu.VMEM((_CPW, _K, _CM), jnp.int32),
            pltpu.VMEM((_CPW, _CM, _DIM), jnp.float32),
            pltpu.VMEM((_CPW, _CM), jnp.float32),
            pltpu.VMEM((4, _DIM, 33), jnp.float32),
            pltpu.SemaphoreType.DMA,
            pltpu.SemaphoreType.DMA((_CPW,)),
            pltpu.SemaphoreType.DMA,
        ],
    )(scaled_table, x_flat)


def kernel(x, table):
    scaled = _scale_table(table).reshape(-1, _DIM)
    out = _sc_embed(scaled, x.transpose(1, 2, 0))
    return out.transpose(2, 0, 1)
